# baseline (speedup 1.0000x reference)
import functools

import jax
import jax.numpy as jnp
from jax import lax
from jax.experimental import pallas as pl
from jax.experimental.pallas import tpu as pltpu
from jax.experimental.pallas import tpu_sc as plsc

BATCH = 16384
EMB = 64
NC = 2
NS = 16
NW = NC * NS
BPW = BATCH // NW
L = 16


def _body(u_hbm, i_hbm, ue_hbm, ie_hbm, ub_hbm, ib_hbm, gb_hbm, out_hbm,
          u_idx, i_idx, u_rows, i_rows, u_b, i_b, gb_v, out_v,
          sem0, sem1, sem2, sem3, sem4):
    wid = lax.axis_index("s") * NC + lax.axis_index("c")
    base = wid * BPW

    pltpu.sync_copy(u_hbm.at[pl.ds(base, BPW)], u_idx)
    pltpu.sync_copy(i_hbm.at[pl.ds(base, BPW)], i_idx)
    pltpu.sync_copy(gb_hbm, gb_v)

    cp0 = pltpu.async_copy(ue_hbm.at[u_idx], u_rows, sem0)
    cp1 = pltpu.async_copy(ie_hbm.at[i_idx], i_rows, sem1)
    cp2 = pltpu.async_copy(ub_hbm.at[u_idx], u_b, sem2)
    cp3 = pltpu.async_copy(ib_hbm.at[i_idx], i_b, sem3)
    cp0.wait()
    cp1.wait()
    cp2.wait()
    cp3.wait()

    gb = gb_v[...]
    iota = lax.iota(jnp.int32, L)

    def group(g, carry):
        rbase = g * L
        row_idx = rbase + iota
        acc = u_b[pl.ds(rbase, L)] + i_b[pl.ds(rbase, L)] + gb
        for c in range(EMB):
            ci = jnp.full((L,), c, jnp.int32)
            uv = plsc.load_gather(u_rows, [row_idx, ci])
            iv = plsc.load_gather(i_rows, [row_idx, ci])
            acc = acc + uv * iv
        out_v[pl.ds(rbase, L)] = acc
        return carry

    lax.fori_loop(0, BPW // L, group, 0, unroll=False)

    pltpu.sync_copy(out_v, out_hbm.at[pl.ds(base, BPW)])


def kernel(u, i, user_embed, item_embed, user_bias, item_bias, global_bias):
    mesh = plsc.VectorSubcoreMesh(core_axis_name="c", subcore_axis_name="s",
                                  num_cores=NC, num_subcores=NS)
    f = pl.kernel(
        _body,
        out_type=jax.ShapeDtypeStruct((BATCH,), jnp.float32),
        mesh=mesh,
        compiler_params=pltpu.CompilerParams(needs_layout_passes=False,
                                             use_tc_tiling_on_sc=False),
        scratch_types=[
            pltpu.VMEM((BPW,), jnp.int32),
            pltpu.VMEM((BPW,), jnp.int32),
            pltpu.VMEM((BPW, EMB), jnp.float32),
            pltpu.VMEM((BPW, EMB), jnp.float32),
            pltpu.VMEM((BPW,), jnp.float32),
            pltpu.VMEM((BPW,), jnp.float32),
            pltpu.VMEM((L,), jnp.float32),
            pltpu.VMEM((BPW,), jnp.float32),
            pltpu.SemaphoreType.DMA,
            pltpu.SemaphoreType.DMA,
            pltpu.SemaphoreType.DMA,
            pltpu.SemaphoreType.DMA,
            pltpu.SemaphoreType.DMA,
        ],
    )
    gb16 = jnp.broadcast_to(global_bias.astype(jnp.float32), (L,))
    return f(u.astype(jnp.int32), i.astype(jnp.int32),
             user_embed, item_embed,
             user_bias.reshape(-1), item_bias.reshape(-1), gb16)
